# baseline (speedup 1.0000x reference)
import jax
import jax.numpy as jnp
from jax.experimental import pallas as pl
from jax.experimental.pallas import tpu as pltpu

_M = 512
_DS = 256
_BT = 2048


def _selector_kernel(x_ref, Ws0h_ref, Ws0l_ref, bs0_ref, Ws1h_ref, Ws1l_ref,
                     bs1_ref, w_ref):
    t = pl.program_id(0)
    nt = pl.num_programs(0)

    x = x_ref[...]
    h32 = (jnp.dot(x, Ws0h_ref[...], preferred_element_type=jnp.float32)
           + jnp.dot(x, Ws0l_ref[...], preferred_element_type=jnp.float32)
           + bs0_ref[...])
    h = jnp.maximum(h32, 0.0).astype(jnp.bfloat16)
    s = (jnp.dot(h, Ws1h_ref[...], preferred_element_type=jnp.float32)
         + jnp.dot(h, Ws1l_ref[...], preferred_element_type=jnp.float32)
         + bs1_ref[...])
    neg = -s
    m = jnp.max(neg, axis=1, keepdims=True)
    e = jnp.exp(neg - m)
    p = e / jnp.sum(e, axis=1, keepdims=True)
    part = jnp.sum(p, axis=0, keepdims=True)

    @pl.when(t == 0)
    def _():
        w_ref[...] = part

    @pl.when(t != 0)
    def _():
        w_ref[...] = w_ref[...] + part

    @pl.when(t == nt - 1)
    def _():
        w = w_ref[...]
        wr = jnp.broadcast_to(w, (_M, _M))
        wc = wr.T
        i_idx = jax.lax.broadcasted_iota(jnp.int32, (_M, _M), 0)
        j_idx = jax.lax.broadcasted_iota(jnp.int32, (_M, _M), 1)
        gt = (wc > wr).astype(jnp.float32)
        tie = jnp.logical_and(wc == wr, i_idx < j_idx).astype(jnp.float32)
        rank = jnp.sum(gt + tie, axis=0, keepdims=True)
        keep = rank < float(_DS)
        wk = jnp.where(keep, w, 0.0)
        w_ref[...] = wk / jnp.sum(wk)


def _predictor_kernel(x_ref, w_ref, Wp0_ref, bp0_ref, Wp1_ref, bp1_ref, out_ref):
    xw = (x_ref[...].astype(jnp.float32) * w_ref[...]).astype(jnp.bfloat16)
    h32 = (jnp.dot(xw, Wp0_ref[...], preferred_element_type=jnp.float32)
           + bp0_ref[...])
    h = jnp.maximum(h32, 0.0).astype(jnp.bfloat16)
    out_ref[...] = (
        jnp.dot(h, Wp1_ref[...], preferred_element_type=jnp.float32) + bp1_ref[...])


def kernel(x, Ws0, bs0, Ws1, bs1, Wp0, bp0, Wp1, bp1):
    B, M = x.shape
    H = Ws0.shape[1]
    nt = B // _BT

    bs0_2d = bs0.reshape(1, H)
    bs1_2d = bs1.reshape(1, M)
    bp0_2d = bp0.reshape(1, H)
    bp1_2d = bp1.reshape(1, M)

    xb = x.astype(jnp.bfloat16)

    def split(W):
        hi = W.astype(jnp.bfloat16)
        lo = (W - hi.astype(jnp.float32)).astype(jnp.bfloat16)
        return hi, lo

    Ws0h, Ws0l = split(Ws0)
    Ws1h, Ws1l = split(Ws1)
    Wp0b = Wp0.astype(jnp.bfloat16)
    Wp1b = Wp1.astype(jnp.bfloat16)

    w = pl.pallas_call(
        _selector_kernel,
        grid=(nt,),
        in_specs=[
            pl.BlockSpec((_BT, M), lambda t: (t, 0)),
            pl.BlockSpec((M, H), lambda t: (0, 0)),
            pl.BlockSpec((M, H), lambda t: (0, 0)),
            pl.BlockSpec((1, H), lambda t: (0, 0)),
            pl.BlockSpec((H, M), lambda t: (0, 0)),
            pl.BlockSpec((H, M), lambda t: (0, 0)),
            pl.BlockSpec((1, M), lambda t: (0, 0)),
        ],
        out_specs=pl.BlockSpec((1, M), lambda t: (0, 0)),
        out_shape=jax.ShapeDtypeStruct((1, M), jnp.float32),
        compiler_params=pltpu.CompilerParams(
            dimension_semantics=("arbitrary",)),
    )(xb, Ws0h, Ws0l, bs0_2d, Ws1h, Ws1l, bs1_2d)

    out = pl.pallas_call(
        _predictor_kernel,
        grid=(nt,),
        in_specs=[
            pl.BlockSpec((_BT, M), lambda t: (t, 0)),
            pl.BlockSpec((1, M), lambda t: (0, 0)),
            pl.BlockSpec((M, H), lambda t: (0, 0)),
            pl.BlockSpec((1, H), lambda t: (0, 0)),
            pl.BlockSpec((H, M), lambda t: (0, 0)),
            pl.BlockSpec((1, M), lambda t: (0, 0)),
        ],
        out_specs=pl.BlockSpec((_BT, M), lambda t: (t, 0)),
        out_shape=jax.ShapeDtypeStruct((B, M), jnp.float32),
        compiler_params=pltpu.CompilerParams(
            dimension_semantics=("parallel",)),
    )(xb, w, Wp0b, bp0_2d, Wp1b, bp1_2d)

    return out


# speedup vs baseline: 1.1218x; 1.1218x over previous
import jax
import jax.numpy as jnp
from jax.experimental import pallas as pl
from jax.experimental.pallas import tpu as pltpu

_M = 512
_DS = 256
_BT = 2048


def _selector_kernel(x_ref, Ws0c_ref, bs0_ref, Ws1c_ref, bs1_ref, w_ref):
    t = pl.program_id(0)
    nt = pl.num_programs(0)

    xb = x_ref[...].astype(jnp.bfloat16)
    xcat = jnp.concatenate([xb, xb], axis=1)
    h32 = (jnp.dot(xcat, Ws0c_ref[...], preferred_element_type=jnp.float32)
           + bs0_ref[...])
    h = jnp.maximum(h32, 0.0).astype(jnp.bfloat16)
    hcat = jnp.concatenate([h, h], axis=1)
    s = (jnp.dot(hcat, Ws1c_ref[...], preferred_element_type=jnp.float32)
         + bs1_ref[...])
    neg = -s
    m = jnp.max(neg, axis=1, keepdims=True)
    e = jnp.exp(neg - m)
    p = e / jnp.sum(e, axis=1, keepdims=True)
    part = jnp.sum(p, axis=0, keepdims=True)

    @pl.when(t == 0)
    def _():
        w_ref[...] = part

    @pl.when(t != 0)
    def _():
        w_ref[...] = w_ref[...] + part

    @pl.when(t == nt - 1)
    def _():
        w = w_ref[...]
        wr = jnp.broadcast_to(w, (_M, _M))
        wc = wr.T
        i_idx = jax.lax.broadcasted_iota(jnp.int32, (_M, _M), 0)
        j_idx = jax.lax.broadcasted_iota(jnp.int32, (_M, _M), 1)
        gt = (wc > wr).astype(jnp.float32)
        tie = jnp.logical_and(wc == wr, i_idx < j_idx).astype(jnp.float32)
        rank = jnp.sum(gt + tie, axis=0, keepdims=True)
        keep = rank < float(_DS)
        wk = jnp.where(keep, w, 0.0)
        w_ref[...] = wk / jnp.sum(wk)


def _predictor_kernel(x_ref, w_ref, Wp0_ref, bp0_ref, Wp1_ref, bp1_ref, out_ref):
    xw = (x_ref[...] * w_ref[...]).astype(jnp.bfloat16)
    h32 = (jnp.dot(xw, Wp0_ref[...], preferred_element_type=jnp.float32)
           + bp0_ref[...])
    h = jnp.maximum(h32, 0.0).astype(jnp.bfloat16)
    out_ref[...] = (
        jnp.dot(h, Wp1_ref[...], preferred_element_type=jnp.float32) + bp1_ref[...])


def kernel(x, Ws0, bs0, Ws1, bs1, Wp0, bp0, Wp1, bp1):
    B, M = x.shape
    H = Ws0.shape[1]
    nt = B // _BT

    bs0_2d = bs0.reshape(1, H)
    bs1_2d = bs1.reshape(1, M)
    bp0_2d = bp0.reshape(1, H)
    bp1_2d = bp1.reshape(1, M)

    def split_cat(W):
        hi = W.astype(jnp.bfloat16)
        lo = (W - hi.astype(jnp.float32)).astype(jnp.bfloat16)
        return jnp.concatenate([hi, lo], axis=0)

    Ws0c = split_cat(Ws0)
    Ws1c = split_cat(Ws1)
    Wp0b = Wp0.astype(jnp.bfloat16)
    Wp1b = Wp1.astype(jnp.bfloat16)

    w = pl.pallas_call(
        _selector_kernel,
        grid=(nt,),
        in_specs=[
            pl.BlockSpec((_BT, M), lambda t: (t, 0)),
            pl.BlockSpec((2 * M, H), lambda t: (0, 0)),
            pl.BlockSpec((1, H), lambda t: (0, 0)),
            pl.BlockSpec((2 * H, M), lambda t: (0, 0)),
            pl.BlockSpec((1, M), lambda t: (0, 0)),
        ],
        out_specs=pl.BlockSpec((1, M), lambda t: (0, 0)),
        out_shape=jax.ShapeDtypeStruct((1, M), jnp.float32),
        compiler_params=pltpu.CompilerParams(
            dimension_semantics=("arbitrary",)),
    )(x, Ws0c, bs0_2d, Ws1c, bs1_2d)

    out = pl.pallas_call(
        _predictor_kernel,
        grid=(nt,),
        in_specs=[
            pl.BlockSpec((_BT, M), lambda t: (t, 0)),
            pl.BlockSpec((1, M), lambda t: (0, 0)),
            pl.BlockSpec((M, H), lambda t: (0, 0)),
            pl.BlockSpec((1, H), lambda t: (0, 0)),
            pl.BlockSpec((H, M), lambda t: (0, 0)),
            pl.BlockSpec((1, M), lambda t: (0, 0)),
        ],
        out_specs=pl.BlockSpec((_BT, M), lambda t: (t, 0)),
        out_shape=jax.ShapeDtypeStruct((B, M), jnp.float32),
        compiler_params=pltpu.CompilerParams(
            dimension_semantics=("parallel",)),
    )(x, w, Wp0b, bp0_2d, Wp1b, bp1_2d)

    return out


# speedup vs baseline: 1.6192x; 1.4433x over previous
import jax
import jax.numpy as jnp
from jax.experimental import pallas as pl
from jax.experimental.pallas import tpu as pltpu

_M = 512
_DS = 256
_BT = 2048


def _selector_kernel(x_ref, Ws0_ref, bs0_ref, Ws1_ref, bs1_ref, part_ref):
    x = x_ref[...]
    h = jnp.maximum(
        jnp.dot(x, Ws0_ref[...], preferred_element_type=jnp.float32)
        + bs0_ref[...], 0.0)
    s = (jnp.dot(h, Ws1_ref[...], preferred_element_type=jnp.float32)
         + bs1_ref[...])
    neg = -s
    m = jnp.max(neg, axis=1, keepdims=True)
    e = jnp.exp(neg - m)
    p = e / jnp.sum(e, axis=1, keepdims=True)
    part_ref[...] = jnp.sum(p, axis=0, keepdims=True).reshape(1, 1, -1)


def _finalize_kernel(part_ref, w_ref):
    w = jnp.sum(part_ref[...], axis=0)
    wr = jnp.broadcast_to(w, (_M, _M))
    wc = wr.T
    i_idx = jax.lax.broadcasted_iota(jnp.int32, (_M, _M), 0)
    j_idx = jax.lax.broadcasted_iota(jnp.int32, (_M, _M), 1)
    gt = (wc > wr).astype(jnp.float32)
    tie = jnp.logical_and(wc == wr, i_idx < j_idx).astype(jnp.float32)
    rank = jnp.sum(gt + tie, axis=0, keepdims=True)
    keep = rank < float(_DS)
    wk = jnp.where(keep, w, 0.0)
    w_ref[...] = wk / jnp.sum(wk)


def _predictor_kernel(x_ref, w_ref, Wp0_ref, bp0_ref, Wp1_ref, bp1_ref, out_ref):
    xw = x_ref[...] * w_ref[...]
    h = jnp.maximum(
        jnp.dot(xw, Wp0_ref[...], preferred_element_type=jnp.float32)
        + bp0_ref[...], 0.0)
    out_ref[...] = (
        jnp.dot(h, Wp1_ref[...], preferred_element_type=jnp.float32)
        + bp1_ref[...])


def kernel(x, Ws0, bs0, Ws1, bs1, Wp0, bp0, Wp1, bp1):
    B, M = x.shape
    H = Ws0.shape[1]
    nt = B // _BT

    bs0_2d = bs0.reshape(1, H)
    bs1_2d = bs1.reshape(1, M)
    bp0_2d = bp0.reshape(1, H)
    bp1_2d = bp1.reshape(1, M)

    parts = pl.pallas_call(
        _selector_kernel,
        grid=(nt,),
        in_specs=[
            pl.BlockSpec((_BT, M), lambda t: (t, 0)),
            pl.BlockSpec((M, H), lambda t: (0, 0)),
            pl.BlockSpec((1, H), lambda t: (0, 0)),
            pl.BlockSpec((H, M), lambda t: (0, 0)),
            pl.BlockSpec((1, M), lambda t: (0, 0)),
        ],
        out_specs=pl.BlockSpec((1, 1, M), lambda t: (t, 0, 0)),
        out_shape=jax.ShapeDtypeStruct((nt, 1, M), jnp.float32),
        compiler_params=pltpu.CompilerParams(
            dimension_semantics=("parallel",)),
    )(x, Ws0, bs0_2d, Ws1, bs1_2d)

    w = pl.pallas_call(
        _finalize_kernel,
        in_specs=[pl.BlockSpec((nt, 1, M), lambda: (0, 0, 0))],
        out_specs=pl.BlockSpec((1, M), lambda: (0, 0)),
        out_shape=jax.ShapeDtypeStruct((1, M), jnp.float32),
    )(parts)

    out = pl.pallas_call(
        _predictor_kernel,
        grid=(nt,),
        in_specs=[
            pl.BlockSpec((_BT, M), lambda t: (t, 0)),
            pl.BlockSpec((1, M), lambda t: (0, 0)),
            pl.BlockSpec((M, H), lambda t: (0, 0)),
            pl.BlockSpec((1, H), lambda t: (0, 0)),
            pl.BlockSpec((H, M), lambda t: (0, 0)),
            pl.BlockSpec((1, M), lambda t: (0, 0)),
        ],
        out_specs=pl.BlockSpec((_BT, M), lambda t: (t, 0)),
        out_shape=jax.ShapeDtypeStruct((B, M), jnp.float32),
        compiler_params=pltpu.CompilerParams(
            dimension_semantics=("parallel",)),
    )(x, w, Wp0, bp0_2d, Wp1, bp1_2d)

    return out


# speedup vs baseline: 1.6568x; 1.0232x over previous
import jax
import jax.numpy as jnp
from jax.experimental import pallas as pl
from jax.experimental.pallas import tpu as pltpu

_M = 512
_DS = 256
_BT = 1024


def _selector_kernel(x_ref, Ws0_ref, bs0_ref, Ws1_ref, bs1_ref, part_ref):
    x = x_ref[...]
    h = jnp.maximum(
        jnp.dot(x, Ws0_ref[...], preferred_element_type=jnp.float32)
        + bs0_ref[...], 0.0)
    s = (jnp.dot(h, Ws1_ref[...], preferred_element_type=jnp.float32)
         + bs1_ref[...])
    e = jnp.exp(-s)
    p = e / jnp.sum(e, axis=1, keepdims=True)
    part_ref[...] = jnp.sum(p, axis=0, keepdims=True).reshape(1, 1, -1)


def _finalize_kernel(part_ref, w_ref):
    w = jnp.sum(part_ref[...], axis=0)
    wr = jnp.broadcast_to(w, (_M, _M))
    wc = wr.T
    i_idx = jax.lax.broadcasted_iota(jnp.int32, (_M, _M), 0)
    j_idx = jax.lax.broadcasted_iota(jnp.int32, (_M, _M), 1)
    gt = (wc > wr).astype(jnp.float32)
    tie = jnp.logical_and(wc == wr, i_idx < j_idx).astype(jnp.float32)
    rank = jnp.sum(gt + tie, axis=0, keepdims=True)
    keep = rank < float(_DS)
    wk = jnp.where(keep, w, 0.0)
    w_ref[...] = wk / jnp.sum(wk)


def _predictor_kernel(x_ref, w_ref, Wp0_ref, bp0_ref, Wp1_ref, bp1_ref, out_ref):
    xw = x_ref[...] * w_ref[...]
    h = jnp.maximum(
        jnp.dot(xw, Wp0_ref[...], preferred_element_type=jnp.float32)
        + bp0_ref[...], 0.0)
    out_ref[...] = (
        jnp.dot(h, Wp1_ref[...], preferred_element_type=jnp.float32)
        + bp1_ref[...])


def kernel(x, Ws0, bs0, Ws1, bs1, Wp0, bp0, Wp1, bp1):
    B, M = x.shape
    H = Ws0.shape[1]
    nt = B // _BT

    bs0_2d = bs0.reshape(1, H)
    bs1_2d = bs1.reshape(1, M)
    bp0_2d = bp0.reshape(1, H)
    bp1_2d = bp1.reshape(1, M)

    parts = pl.pallas_call(
        _selector_kernel,
        grid=(nt,),
        in_specs=[
            pl.BlockSpec((_BT, M), lambda t: (t, 0)),
            pl.BlockSpec((M, H), lambda t: (0, 0)),
            pl.BlockSpec((1, H), lambda t: (0, 0)),
            pl.BlockSpec((H, M), lambda t: (0, 0)),
            pl.BlockSpec((1, M), lambda t: (0, 0)),
        ],
        out_specs=pl.BlockSpec((1, 1, M), lambda t: (t, 0, 0)),
        out_shape=jax.ShapeDtypeStruct((nt, 1, M), jnp.float32),
        compiler_params=pltpu.CompilerParams(
            dimension_semantics=("parallel",)),
    )(x, Ws0, bs0_2d, Ws1, bs1_2d)

    w = pl.pallas_call(
        _finalize_kernel,
        in_specs=[pl.BlockSpec((nt, 1, M), lambda: (0, 0, 0))],
        out_specs=pl.BlockSpec((1, M), lambda: (0, 0)),
        out_shape=jax.ShapeDtypeStruct((1, M), jnp.float32),
    )(parts)

    out = pl.pallas_call(
        _predictor_kernel,
        grid=(nt,),
        in_specs=[
            pl.BlockSpec((_BT, M), lambda t: (t, 0)),
            pl.BlockSpec((1, M), lambda t: (0, 0)),
            pl.BlockSpec((M, H), lambda t: (0, 0)),
            pl.BlockSpec((1, H), lambda t: (0, 0)),
            pl.BlockSpec((H, M), lambda t: (0, 0)),
            pl.BlockSpec((1, M), lambda t: (0, 0)),
        ],
        out_specs=pl.BlockSpec((_BT, M), lambda t: (t, 0)),
        out_shape=jax.ShapeDtypeStruct((B, M), jnp.float32),
        compiler_params=pltpu.CompilerParams(
            dimension_semantics=("parallel",)),
    )(x, w, Wp0, bp0_2d, Wp1, bp1_2d)

    return out
